# baseline (speedup 1.0000x reference)
import functools

import jax
import jax.numpy as jnp
from jax import lax
from jax.experimental import pallas as pl
from jax.experimental.pallas import tpu as pltpu
from jax.experimental.pallas import tpu_sc as plsc

NC, NS = 2, 16
NW = NC * NS
R = 2
NBUF = 2
DP = 128


def _gather(idx, table):
    n_in, T = idx.shape
    D = 64
    rows_per_w = n_in // NW
    n_chunks = rows_per_w // R
    SPLIT = 128

    mesh = plsc.VectorSubcoreMesh(
        core_axis_name="c", subcore_axis_name="s",
        num_cores=NC, num_subcores=NS)

    @functools.partial(
        pl.kernel,
        out_type=jax.ShapeDtypeStruct((n_in, T, D), jnp.float32),
        mesh=mesh,
        scratch_types=(
            [pltpu.VMEM((R, T), jnp.int32) for _ in range(NBUF)]
            + [pltpu.VMEM((R, T, DP), jnp.float32) for _ in range(NBUF)]
            + [pltpu.SemaphoreType.DMA for _ in range(3 * NBUF)]
        ),
        compiler_params=pltpu.CompilerParams(use_tc_tiling_on_sc=False),
    )
    def k(idx_hbm, table_hbm, out_hbm, *refs):
        idx_v = refs[:NBUF]
        rows = refs[NBUF:2 * NBUF]
        si = refs[2 * NBUF:3 * NBUF]
        sg = refs[3 * NBUF:4 * NBUF]
        so = refs[4 * NBUF:]
        wid = lax.axis_index("s") * NC + lax.axis_index("c")
        row0 = wid * rows_per_w

        def idx_pair(g, b):
            return (idx_hbm.at[pl.ds(row0 + g * R, R)], idx_v[b], si[b])

        def out_pair(g, b):
            return (rows[b].at[:, :, pl.ds(0, D)],
                    out_hbm.at[pl.ds(row0 + g * R, R)], so[b])

        def fire_idx(g, b):
            pltpu.async_copy(*idx_pair(g, b))

        def wait_idx(g, b):
            pltpu.make_async_copy(*idx_pair(g, b)).wait()

        def fire_gather(b):
            for i in range(R):
                pltpu.async_copy(
                    table_hbm.at[idx_v[b].at[i, pl.ds(0, SPLIT)]],
                    rows[b].at[i, pl.ds(0, SPLIT)],
                    sg[b])
                pltpu.async_copy(
                    table_hbm.at[idx_v[b].at[i, pl.ds(SPLIT, T - SPLIT)]],
                    rows[b].at[i, pl.ds(SPLIT, T - SPLIT)],
                    sg[b])

        def wait_gather(b):
            for i in range(R):
                pltpu.make_async_copy(
                    table_hbm.at[idx_v[b].at[i, pl.ds(0, SPLIT)]],
                    rows[b].at[i, pl.ds(0, SPLIT)],
                    sg[b]).wait()
                pltpu.make_async_copy(
                    table_hbm.at[idx_v[b].at[i, pl.ds(SPLIT, T - SPLIT)]],
                    rows[b].at[i, pl.ds(SPLIT, T - SPLIT)],
                    sg[b]).wait()

        def start_out(g, b):
            pltpu.async_copy(*out_pair(g, b))

        def wait_out(g, b):
            pltpu.make_async_copy(*out_pair(g, b)).wait()

        fire_idx(0, 0)
        wait_idx(0, 0)
        fire_gather(0)
        fire_idx(1, 1)

        def step(g, b):
            nb = 1 - b

            @pl.when(g + 1 < n_chunks)
            def _():
                wait_idx(g + 1, nb)

                @pl.when(g >= 1)
                def _():
                    wait_out(g - 1, nb)

                fire_gather(nb)

            wait_gather(b)
            start_out(g, b)

            @pl.when(g + 2 < n_chunks)
            def _():
                fire_idx(g + 2, b)

        @pl.loop(0, n_chunks, step=NBUF)
        def outer(t):
            for b in range(NBUF):
                step(t + b, b)

        wait_out(n_chunks - 2, (n_chunks - 2) % NBUF)
        wait_out(n_chunks - 1, (n_chunks - 1) % NBUF)

    return k(idx, table)


def kernel(input, weight):
    wpad = jnp.pad(weight, ((0, 0), (0, DP - weight.shape[1])))
    return _gather(input.astype(jnp.int32), wpad)


# speedup vs baseline: 1.2584x; 1.2584x over previous
import functools

import jax
import jax.numpy as jnp
from jax import lax
from jax.experimental import pallas as pl
from jax.experimental.pallas import tpu as pltpu
from jax.experimental.pallas import tpu_sc as plsc

NC, NS = 2, 16
NW = NC * NS
R = 2
NBUF = 2
DP = 128


def _gather(idx, table):
    n_in, T = idx.shape
    D = 64
    rows_per_w = n_in // NW
    n_chunks = rows_per_w // R
    SPLIT = 128

    mesh = plsc.VectorSubcoreMesh(
        core_axis_name="c", subcore_axis_name="s",
        num_cores=NC, num_subcores=NS)

    @functools.partial(
        pl.kernel,
        out_type=jax.ShapeDtypeStruct((n_in, T, DP), jnp.float32),
        mesh=mesh,
        scratch_types=(
            [pltpu.VMEM((R, T), jnp.int32) for _ in range(NBUF)]
            + [pltpu.VMEM((R, T, DP), jnp.float32) for _ in range(NBUF)]
            + [pltpu.SemaphoreType.DMA for _ in range(3 * NBUF)]
        ),
        compiler_params=pltpu.CompilerParams(use_tc_tiling_on_sc=False),
    )
    def k(idx_hbm, table_hbm, out_hbm, *refs):
        idx_v = refs[:NBUF]
        rows = refs[NBUF:2 * NBUF]
        si = refs[2 * NBUF:3 * NBUF]
        sg = refs[3 * NBUF:4 * NBUF]
        so = refs[4 * NBUF:]
        wid = lax.axis_index("s") * NC + lax.axis_index("c")
        row0 = wid * rows_per_w

        def idx_pair(g, b):
            return (idx_hbm.at[pl.ds(row0 + g * R, R)], idx_v[b], si[b])

        def out_pair(g, b):
            return (rows[b], out_hbm.at[pl.ds(row0 + g * R, R)], so[b])

        def fire_idx(g, b):
            pltpu.async_copy(*idx_pair(g, b))

        def wait_idx(g, b):
            pltpu.make_async_copy(*idx_pair(g, b)).wait()

        def fire_gather(b):
            for i in range(R):
                pltpu.async_copy(
                    table_hbm.at[idx_v[b].at[i, pl.ds(0, SPLIT)]],
                    rows[b].at[i, pl.ds(0, SPLIT)],
                    sg[b])
                pltpu.async_copy(
                    table_hbm.at[idx_v[b].at[i, pl.ds(SPLIT, T - SPLIT)]],
                    rows[b].at[i, pl.ds(SPLIT, T - SPLIT)],
                    sg[b])

        def wait_gather(b):
            for i in range(R):
                pltpu.make_async_copy(
                    table_hbm.at[idx_v[b].at[i, pl.ds(0, SPLIT)]],
                    rows[b].at[i, pl.ds(0, SPLIT)],
                    sg[b]).wait()
                pltpu.make_async_copy(
                    table_hbm.at[idx_v[b].at[i, pl.ds(SPLIT, T - SPLIT)]],
                    rows[b].at[i, pl.ds(SPLIT, T - SPLIT)],
                    sg[b]).wait()

        def start_out(g, b):
            pltpu.async_copy(*out_pair(g, b))

        def wait_out(g, b):
            pltpu.make_async_copy(*out_pair(g, b)).wait()

        fire_idx(0, 0)
        wait_idx(0, 0)
        fire_gather(0)
        fire_idx(1, 1)

        def step(g, b):
            nb = 1 - b

            @pl.when(g + 1 < n_chunks)
            def _():
                wait_idx(g + 1, nb)

                @pl.when(g >= 1)
                def _():
                    wait_out(g - 1, nb)

                fire_gather(nb)

            wait_gather(b)
            start_out(g, b)

            @pl.when(g + 2 < n_chunks)
            def _():
                fire_idx(g + 2, b)

        @pl.loop(0, n_chunks, step=NBUF)
        def outer(t):
            for b in range(NBUF):
                step(t + b, b)

        wait_out(n_chunks - 2, (n_chunks - 2) % NBUF)
        wait_out(n_chunks - 1, (n_chunks - 1) % NBUF)

    return k(idx, table)


def kernel(input, weight):
    wpad = jnp.pad(weight, ((0, 0), (0, DP - weight.shape[1])))
    out = _gather(input.astype(jnp.int32), wpad)
    return out[:, :, :weight.shape[1]]


# speedup vs baseline: 1.3184x; 1.0477x over previous
import functools

import jax
import jax.numpy as jnp
from jax import lax
from jax.experimental import pallas as pl
from jax.experimental.pallas import tpu as pltpu
from jax.experimental.pallas import tpu_sc as plsc

NC, NS = 2, 16
NW = NC * NS
R = 2
NBUF = 2
DP = 128


def _gather(idx, table):
    n_in, T = idx.shape
    D = 64
    rows_per_w = n_in // NW
    n_chunks = rows_per_w // R
    SPLIT = 128

    mesh = plsc.VectorSubcoreMesh(
        core_axis_name="c", subcore_axis_name="s",
        num_cores=NC, num_subcores=NS)

    @functools.partial(
        pl.kernel,
        out_type=jax.ShapeDtypeStruct((n_in, T, DP), jnp.float32),
        mesh=mesh,
        scratch_types=(
            [pltpu.VMEM((R, T), jnp.int32) for _ in range(NBUF)]
            + [pltpu.VMEM((R, T, DP), jnp.float32) for _ in range(NBUF)]
            + [pltpu.SemaphoreType.DMA for _ in range(3 * NBUF)]
        ),
        compiler_params=pltpu.CompilerParams(use_tc_tiling_on_sc=False),
    )
    def k(idx_hbm, table_hbm, out_hbm, *refs):
        idx_v = refs[:NBUF]
        rows = refs[NBUF:2 * NBUF]
        si = refs[2 * NBUF:3 * NBUF]
        sg = refs[3 * NBUF:4 * NBUF]
        so = refs[4 * NBUF:]
        wid = lax.axis_index("s") * NC + lax.axis_index("c")
        row0 = wid * rows_per_w

        def idx_pair(g, b):
            return (idx_hbm.at[pl.ds(row0 + g * R, R)], idx_v[b], si[b])

        def out_pair(g, b):
            return (rows[b].at[:, :, pl.ds(0, D)],
                    out_hbm.at[pl.ds(row0 + g * R, R), :, pl.ds(0, D)], so[b])

        def fire_idx(g, b):
            pltpu.async_copy(*idx_pair(g, b))

        def wait_idx(g, b):
            pltpu.make_async_copy(*idx_pair(g, b)).wait()

        def fire_gather(b):
            for i in range(R):
                pltpu.async_copy(
                    table_hbm.at[idx_v[b].at[i, pl.ds(0, SPLIT)]],
                    rows[b].at[i, pl.ds(0, SPLIT)],
                    sg[b])
                pltpu.async_copy(
                    table_hbm.at[idx_v[b].at[i, pl.ds(SPLIT, T - SPLIT)]],
                    rows[b].at[i, pl.ds(SPLIT, T - SPLIT)],
                    sg[b])

        def wait_gather(b):
            for i in range(R):
                pltpu.make_async_copy(
                    table_hbm.at[idx_v[b].at[i, pl.ds(0, SPLIT)]],
                    rows[b].at[i, pl.ds(0, SPLIT)],
                    sg[b]).wait()
                pltpu.make_async_copy(
                    table_hbm.at[idx_v[b].at[i, pl.ds(SPLIT, T - SPLIT)]],
                    rows[b].at[i, pl.ds(SPLIT, T - SPLIT)],
                    sg[b]).wait()

        def start_out(g, b):
            pltpu.async_copy(*out_pair(g, b))

        def wait_out(g, b):
            pltpu.make_async_copy(*out_pair(g, b)).wait()

        fire_idx(0, 0)
        wait_idx(0, 0)
        fire_gather(0)
        fire_idx(1, 1)

        def step(g, b):
            nb = 1 - b

            @pl.when(g + 1 < n_chunks)
            def _():
                wait_idx(g + 1, nb)

                @pl.when(g >= 1)
                def _():
                    wait_out(g - 1, nb)

                fire_gather(nb)

            wait_gather(b)
            start_out(g, b)

            @pl.when(g + 2 < n_chunks)
            def _():
                fire_idx(g + 2, b)

        @pl.loop(0, n_chunks, step=NBUF)
        def outer(t):
            for b in range(NBUF):
                step(t + b, b)

        wait_out(n_chunks - 2, (n_chunks - 2) % NBUF)
        wait_out(n_chunks - 1, (n_chunks - 1) % NBUF)

    return k(idx, table)


def kernel(input, weight):
    wpad = jnp.pad(weight, ((0, 0), (0, DP - weight.shape[1])))
    out = _gather(input.astype(jnp.int32), wpad)
    return out[:, :, :weight.shape[1]]


# speedup vs baseline: 1.3742x; 1.0423x over previous
import functools

import jax
import jax.numpy as jnp
from jax import lax
from jax.experimental import pallas as pl
from jax.experimental.pallas import tpu as pltpu
from jax.experimental.pallas import tpu_sc as plsc

NC, NS = 2, 16
NW = NC * NS
R = 2
NBUF = 2
DP = 128


def _gather(idx, table):
    n_in, T = idx.shape
    D = 64
    rows_per_w = n_in // NW
    n_chunks = rows_per_w // R
    SPLIT = 128

    mesh = plsc.VectorSubcoreMesh(
        core_axis_name="c", subcore_axis_name="s",
        num_cores=NC, num_subcores=NS)

    @functools.partial(
        pl.kernel,
        out_type=jax.ShapeDtypeStruct((n_in, T, DP), jnp.float32),
        mesh=mesh,
        scratch_types=(
            [pltpu.VMEM((R, T), jnp.int32) for _ in range(NBUF)]
            + [pltpu.VMEM((R, T, D), jnp.float32) for _ in range(NBUF)]
            + [pltpu.SemaphoreType.DMA for _ in range(3 * NBUF)]
        ),
        compiler_params=pltpu.CompilerParams(use_tc_tiling_on_sc=False),
    )
    def k(idx_hbm, table_hbm, out_hbm, *refs):
        idx_v = refs[:NBUF]
        rows = refs[NBUF:2 * NBUF]
        si = refs[2 * NBUF:3 * NBUF]
        sg = refs[3 * NBUF:4 * NBUF]
        so = refs[4 * NBUF:]
        wid = lax.axis_index("s") * NC + lax.axis_index("c")
        row0 = wid * rows_per_w

        def idx_pair(g, b):
            return (idx_hbm.at[pl.ds(row0 + g * R, R)], idx_v[b], si[b])

        def out_pair(g, b):
            return (rows[b],
                    out_hbm.at[pl.ds(row0 + g * R, R), :, pl.ds(0, D)], so[b])

        def fire_idx(g, b):
            pltpu.async_copy(*idx_pair(g, b))

        def wait_idx(g, b):
            pltpu.make_async_copy(*idx_pair(g, b)).wait()

        def fire_gather(b):
            for i in range(R):
                pltpu.async_copy(
                    table_hbm.at[idx_v[b].at[i, pl.ds(0, SPLIT)]],
                    rows[b].at[i, pl.ds(0, SPLIT)],
                    sg[b])
                pltpu.async_copy(
                    table_hbm.at[idx_v[b].at[i, pl.ds(SPLIT, T - SPLIT)]],
                    rows[b].at[i, pl.ds(SPLIT, T - SPLIT)],
                    sg[b])

        def wait_gather(b):
            for i in range(R):
                pltpu.make_async_copy(
                    table_hbm.at[idx_v[b].at[i, pl.ds(0, SPLIT)]],
                    rows[b].at[i, pl.ds(0, SPLIT)],
                    sg[b]).wait()
                pltpu.make_async_copy(
                    table_hbm.at[idx_v[b].at[i, pl.ds(SPLIT, T - SPLIT)]],
                    rows[b].at[i, pl.ds(SPLIT, T - SPLIT)],
                    sg[b]).wait()

        def start_out(g, b):
            pltpu.async_copy(*out_pair(g, b))

        def wait_out(g, b):
            pltpu.make_async_copy(*out_pair(g, b)).wait()

        fire_idx(0, 0)
        wait_idx(0, 0)
        fire_gather(0)
        fire_idx(1, 1)

        def step(g, b):
            nb = 1 - b

            @pl.when(g + 1 < n_chunks)
            def _():
                wait_idx(g + 1, nb)

                @pl.when(g >= 1)
                def _():
                    wait_out(g - 1, nb)

                fire_gather(nb)

            wait_gather(b)
            start_out(g, b)

            @pl.when(g + 2 < n_chunks)
            def _():
                fire_idx(g + 2, b)

        @pl.loop(0, n_chunks, step=NBUF)
        def outer(t):
            for b in range(NBUF):
                step(t + b, b)

        wait_out(n_chunks - 2, (n_chunks - 2) % NBUF)
        wait_out(n_chunks - 1, (n_chunks - 1) % NBUF)

    return k(idx, table)


def kernel(input, weight):
    out = _gather(input.astype(jnp.int32), weight)
    return out[:, :, :weight.shape[1]]
